# Initial kernel scaffold; baseline (speedup 1.0000x reference)
import functools

import jax
import jax.numpy as jnp
from jax import lax
from jax.experimental import pallas as pl
from jax.experimental.pallas import tpu as pltpu
from jax.experimental.pallas import tpu_sc as plsc

L = 16
CHUNK = 128
NC = 2
NS = 16
NW = NC * NS


def _rsqrt16(d):
    i = plsc.bitcast(d, jnp.int32)
    one = jnp.full((16,), 1, jnp.int32)
    i = jnp.full((16,), 0x5F3759DF, jnp.int32) - lax.shift_right_logical(i, one)
    y = plsc.bitcast(i, jnp.float32)
    for _ in range(3):
        y = y * (1.5 - 0.5 * d * y * y)
    return y


def _make_sc_kernel(N, D, n_chunks):
    NP = ((N + NW * 8 - 1) // (NW * 8)) * (NW * 8)
    slice_rows = NP // NS
    deg_cpt = n_chunks // NS
    edge_cpt = n_chunks // NW
    assert slice_rows % CHUNK == 0
    mesh = plsc.VectorSubcoreMesh(core_axis_name="c", subcore_axis_name="s")

    @functools.partial(
        pl.kernel,
        out_type=jax.ShapeDtypeStruct((NC, NP, D), jnp.float32),
        mesh=mesh,
        scratch_types=[
            pltpu.VMEM((NP,), jnp.float32),
            pltpu.VMEM((NP // NS, L), jnp.float32),
            pltpu.VMEM((NP // NS,), jnp.float32),
            pltpu.VMEM((CHUNK,), jnp.int32),
            pltpu.VMEM((1, CHUNK), jnp.int32),
            pltpu.VMEM((1, CHUNK), jnp.float32),
            pltpu.VMEM((CHUNK,), jnp.float32),
            pltpu.VMEM((CHUNK, L), jnp.float32),
            pltpu.VMEM((CHUNK, D), jnp.float32),
            pltpu.VMEM_SHARED((NP, D), jnp.float32),
            pltpu.VMEM_SHARED((NP, L), jnp.float32),
            pltpu.VMEM_SHARED((NP,), jnp.float32),
            pltpu.SemaphoreType.DMA,
        ],
    )
    def sc_kernel(h_hbm, rowh, colh, ewh, out,
                  dis_v, degv, dis_sl, row_v, col_v, ew_v, w_v, wsrc,
                  rows_v, S_sp, deg_sp, dis_sp, gsem):
        cid = lax.axis_index("c")
        sid = lax.axis_index("s")
        base = sid * slice_rows
        zero16 = jnp.zeros((16,), jnp.float32)

        def zrow(r, _):
            wsrc[r, :] = zero16
            for j in range(D // L):
                rows_v[r, pl.ds(L * j, L)] = zero16
            return 0
        lax.fori_loop(0, CHUNK, zrow, 0)
        for i in range(slice_rows // CHUNK):
            pltpu.sync_copy(wsrc, deg_sp.at[pl.ds(base + CHUNK * i, CHUNK)])
            pltpu.sync_copy(rows_v, S_sp.at[pl.ds(base + CHUNK * i, CHUNK)])
        plsc.subcore_barrier()

        def deg_chunk(c, _):
            ck = sid * deg_cpt + c
            pltpu.sync_copy(colh.at[ck], col_v.at[0])
            pltpu.sync_copy(ewh.at[ck], ew_v.at[0])

            def splat_row(r, _):
                w = ew_v[0, r]
                wsrc[r, :] = jnp.full((16,), w, jnp.float32)
                return 0
            lax.fori_loop(0, CHUNK, splat_row, 0)
            pltpu.sync_copy(wsrc, deg_sp.at[col_v.at[0]], add=True)
            return 0
        lax.fori_loop(0, deg_cpt, deg_chunk, 0)
        plsc.subcore_barrier()

        pltpu.sync_copy(deg_sp.at[pl.ds(base, slice_rows)], degv)
        lanes = lax.iota(jnp.int32, 16)

        def dis_blk(k, _):
            d = plsc.load_gather(degv, [k * 16 + lanes, lanes])
            y = _rsqrt16(d)
            dis_sl[pl.ds(k * 16, 16)] = jnp.where(d > 0, y, 0.0)
            return 0
        lax.fori_loop(0, slice_rows // 16, dis_blk, 0)
        pltpu.sync_copy(dis_sl, dis_sp.at[pl.ds(base, slice_rows)])
        plsc.subcore_barrier()
        pltpu.sync_copy(dis_sp, dis_v)

        def edge_chunk(c, _):
            ck = (cid * NS + sid) * edge_cpt + c
            pltpu.sync_copy(rowh.at[ck], row_v)
            pltpu.sync_copy(colh.at[ck], col_v.at[0])
            pltpu.sync_copy(ewh.at[ck], ew_v.at[0])
            pltpu.async_copy(h_hbm.at[row_v], rows_v, gsem).wait()
            for s in range(CHUNK // 16):
                r16 = row_v[pl.ds(16 * s, 16)]
                c16 = col_v[0, pl.ds(16 * s, 16)]
                e16 = ew_v[0, pl.ds(16 * s, 16)]
                w = plsc.load_gather(dis_v, [r16]) * e16 * \
                    plsc.load_gather(dis_v, [c16])
                w_v[pl.ds(16 * s, 16)] = w

            def scale_row(r, _):
                ws = w_v[r]
                for j in range(D // L):
                    rows_v[r, pl.ds(L * j, L)] = rows_v[r, pl.ds(L * j, L)] * ws
                return 0
            lax.fori_loop(0, CHUNK, scale_row, 0)
            pltpu.sync_copy(rows_v, S_sp.at[col_v.at[0]], add=True)
            return 0
        lax.fori_loop(0, edge_cpt, edge_chunk, 0)
        plsc.subcore_barrier()

        pltpu.sync_copy(S_sp.at[pl.ds(base, slice_rows)],
                        out.at[cid, pl.ds(base, slice_rows)])

    return sc_kernel, NP


def _mm_kernel(x_ref, w_ref, o_ref):
    o_ref[...] = jnp.dot(x_ref[...], w_ref[...],
                         preferred_element_type=jnp.float32)


def _bn_kernel(N, s_ref, x_ref, b_ref, g_ref, be_ref, o_ref):
    agg = s_ref[0, :N, :] + s_ref[1, :N, :] + b_ref[...]
    mean = jnp.mean(agg, axis=0, keepdims=True)
    var = jnp.mean((agg - mean) ** 2, axis=0, keepdims=True)
    bn = (agg - mean) * lax.rsqrt(var + 1e-5) * g_ref[...] + be_ref[...]
    o_ref[...] = jnp.maximum(bn, 0.0) + x_ref[...]


def kernel(x, edge_index, edge_weight, W, bias, gamma, beta):
    N, D = x.shape
    E = edge_weight.shape[0]
    EA = E + N
    n_chunks = ((EA + CHUNK * NW - 1) // (CHUNK * NW)) * NW
    EP = n_chunks * CHUNK
    pad = EP - EA

    loop = jnp.arange(N, dtype=edge_index.dtype)
    zpad_i = jnp.zeros((pad,), edge_index.dtype)
    rowa = jnp.concatenate([edge_index[0], loop, zpad_i]).reshape(n_chunks, CHUNK)
    cola = jnp.concatenate([edge_index[1], loop, zpad_i]).reshape(n_chunks, CHUNK)
    ewa = jnp.concatenate([edge_weight, jnp.ones((N,), x.dtype),
                           jnp.zeros((pad,), x.dtype)]).reshape(n_chunks, CHUNK)

    RB = 1000
    h = pl.pallas_call(
        _mm_kernel,
        grid=(N // RB,),
        in_specs=[pl.BlockSpec((RB, D), lambda i: (i, 0)),
                  pl.BlockSpec((D, D), lambda i: (0, 0))],
        out_specs=pl.BlockSpec((RB, D), lambda i: (i, 0)),
        out_shape=jax.ShapeDtypeStruct((N, D), jnp.float32),
    )(x, W)

    sc_kernel, NP = _make_sc_kernel(N, D, n_chunks)
    S = sc_kernel(h, rowa, cola, ewa)

    out = pl.pallas_call(
        functools.partial(_bn_kernel, N),
        out_shape=jax.ShapeDtypeStruct((N, D), jnp.float32),
    )(S, x, bias[None, :], gamma[None, :], beta[None, :])
    return out


# speedup vs baseline: 11.2762x; 11.2762x over previous
import functools

import jax
import jax.numpy as jnp
from jax import lax
from jax.experimental import pallas as pl
from jax.experimental.pallas import tpu as pltpu
from jax.experimental.pallas import tpu_sc as plsc

L = 16
CHUNK = 128
NC = 2
NS = 16
NW = NC * NS

_SC_PARAMS = pltpu.CompilerParams(needs_layout_passes=False)


def _make_deg_kernel(NP, n_chunks):
    DW = 128
    slice_rows = NP // NS
    cpt = n_chunks // NW
    mesh = plsc.VectorSubcoreMesh(core_axis_name="c", subcore_axis_name="s", num_cores=NC, num_subcores=NS)

    @functools.partial(
        pl.kernel,
        out_type=jax.ShapeDtypeStruct((NC, NP, DW), jnp.float32),
        mesh=mesh,
        compiler_params=_SC_PARAMS,
        scratch_types=[
            pltpu.VMEM((1, CHUNK), jnp.int32),
            pltpu.VMEM((1, CHUNK), jnp.float32),
            pltpu.VMEM((CHUNK, DW), jnp.float32),
            pltpu.VMEM_SHARED((NP, DW), jnp.float32),
        ],
    )
    def deg_kernel(colh, ewh, out, col_v, ew_v, wsrc, deg_sp):
        cid = lax.axis_index("c")
        sid = lax.axis_index("s")
        base = sid * slice_rows
        zero16 = jnp.zeros((16,), jnp.float32)

        def zrow(r, _):
            for j in range(DW // L):
                wsrc[r, pl.ds(L * j, L)] = zero16
            return 0
        lax.fori_loop(0, CHUNK, zrow, 0)
        for i in range(slice_rows // CHUNK):
            pltpu.sync_copy(wsrc, deg_sp.at[pl.ds(base + CHUNK * i, CHUNK)])
        plsc.subcore_barrier()

        def deg_chunk(c, _):
            ck = (cid * NS + sid) * cpt + c
            pltpu.sync_copy(colh.at[ck], col_v.at[0])
            pltpu.sync_copy(ewh.at[ck], ew_v.at[0])

            def splat_grp(s, _):
                e16 = ew_v[0, pl.ds(16 * s, 16)]
                for r2 in range(16):
                    wsrc[16 * s + r2, pl.ds(0, L)] = jnp.full(
                        (16,), e16[r2], jnp.float32)
                return 0
            lax.fori_loop(0, CHUNK // 16, splat_grp, 0)
            pltpu.sync_copy(wsrc, deg_sp.at[col_v.at[0]], add=True)
            return 0
        lax.fori_loop(0, cpt, deg_chunk, 0)
        plsc.subcore_barrier()
        pltpu.sync_copy(deg_sp.at[pl.ds(base, slice_rows)],
                        out.at[cid, pl.ds(base, slice_rows)])

    return deg_kernel


def _make_edge_kernel(NP, D, n_chunks):
    slice_rows = NP // NS
    cpt = n_chunks // NW
    mesh = plsc.VectorSubcoreMesh(core_axis_name="c", subcore_axis_name="s", num_cores=NC, num_subcores=NS)

    @functools.partial(
        pl.kernel,
        out_type=jax.ShapeDtypeStruct((NC, NP, D), jnp.float32),
        mesh=mesh,
        compiler_params=_SC_PARAMS,
        scratch_types=[
            pltpu.VMEM((CHUNK,), jnp.int32),
            pltpu.VMEM((1, CHUNK), jnp.int32),
            pltpu.VMEM((1, CHUNK), jnp.float32),
            pltpu.VMEM((CHUNK, D), jnp.float32),
            pltpu.VMEM_SHARED((NP, D), jnp.float32),
            pltpu.SemaphoreType.DMA,
        ],
    )
    def edge_kernel(g_hbm, rowh, colh, ewh, out,
                    row_v, col_v, ew_v, rows_v, S_sp, gsem):
        cid = lax.axis_index("c")
        sid = lax.axis_index("s")
        base = sid * slice_rows
        zero16 = jnp.zeros((16,), jnp.float32)

        def zrow(r, _):
            for j in range(D // L):
                rows_v[r, pl.ds(L * j, L)] = zero16
            return 0
        lax.fori_loop(0, CHUNK, zrow, 0)
        for i in range(slice_rows // CHUNK):
            pltpu.sync_copy(rows_v, S_sp.at[pl.ds(base + CHUNK * i, CHUNK)])
        plsc.subcore_barrier()

        def edge_chunk(c, _):
            ck = (cid * NS + sid) * cpt + c
            pltpu.sync_copy(rowh.at[ck], row_v)
            pltpu.sync_copy(colh.at[ck], col_v.at[0])
            pltpu.sync_copy(ewh.at[ck], ew_v.at[0])
            pltpu.async_copy(g_hbm.at[row_v], rows_v, gsem).wait()

            def scale_grp(s, _):
                e16 = ew_v[0, pl.ds(16 * s, 16)]
                for r2 in range(16):
                    r = 16 * s + r2
                    ws = e16[r2]
                    for j in range(D // L):
                        rows_v[r, pl.ds(L * j, L)] = (
                            rows_v[r, pl.ds(L * j, L)] * ws)
                return 0
            lax.fori_loop(0, CHUNK // 16, scale_grp, 0)
            pltpu.sync_copy(rows_v, S_sp.at[col_v.at[0]], add=True)
            return 0
        lax.fori_loop(0, cpt, edge_chunk, 0)
        plsc.subcore_barrier()
        pltpu.sync_copy(S_sp.at[pl.ds(base, slice_rows)],
                        out.at[cid, pl.ds(base, slice_rows)])

    return edge_kernel


def _mm_kernel(deg_ref, x_ref, w_ref, o_ref):
    d = deg_ref[0, :, 0:1] + deg_ref[1, :, 0:1]
    dis = jnp.where(d > 0, lax.rsqrt(d), 0.0)
    o_ref[...] = dis * jnp.dot(x_ref[...], w_ref[...],
                               preferred_element_type=jnp.float32)


def _bn_kernel(N, s_ref, deg_ref, x_ref, b_ref, g_ref, be_ref, o_ref):
    d = deg_ref[0, :N, 0:1] + deg_ref[1, :N, 0:1]
    dis = jnp.where(d > 0, lax.rsqrt(d), 0.0)
    agg = dis * (s_ref[0, :N, :] + s_ref[1, :N, :]) + b_ref[...]
    mean = jnp.mean(agg, axis=0, keepdims=True)
    var = jnp.mean((agg - mean) ** 2, axis=0, keepdims=True)
    bn = (agg - mean) * lax.rsqrt(var + 1e-5) * g_ref[...] + be_ref[...]
    o_ref[...] = jnp.maximum(bn, 0.0) + x_ref[...]


def kernel(x, edge_index, edge_weight, W, bias, gamma, beta):
    N, D = x.shape
    E = edge_weight.shape[0]
    EA = E + N
    n_chunks = ((EA + CHUNK * NW - 1) // (CHUNK * NW)) * NW
    EP = n_chunks * CHUNK
    pad = EP - EA
    NP = ((N + NW * 8 - 1) // (NW * 8)) * (NW * 8)

    loop = jnp.arange(N, dtype=edge_index.dtype)
    zpad_i = jnp.zeros((pad,), edge_index.dtype)
    rowa = jnp.concatenate([edge_index[0], loop, zpad_i]).reshape(n_chunks,
                                                                  CHUNK)
    cola = jnp.concatenate([edge_index[1], loop, zpad_i]).reshape(n_chunks,
                                                                  CHUNK)
    ewa = jnp.concatenate([edge_weight, jnp.ones((N,), x.dtype),
                           jnp.zeros((pad,), x.dtype)]).reshape(n_chunks,
                                                                CHUNK)

    deg = _make_deg_kernel(NP, n_chunks)(cola, ewa)

    RB = 1000
    g = pl.pallas_call(
        _mm_kernel,
        grid=(N // RB,),
        in_specs=[pl.BlockSpec((NC, RB, 128), lambda i: (0, i, 0)),
                  pl.BlockSpec((RB, D), lambda i: (i, 0)),
                  pl.BlockSpec((D, D), lambda i: (0, 0))],
        out_specs=pl.BlockSpec((RB, D), lambda i: (i, 0)),
        out_shape=jax.ShapeDtypeStruct((N, D), jnp.float32),
    )(deg, x, W)

    S = _make_edge_kernel(NP, D, n_chunks)(g, rowa, cola, ewa)

    out = pl.pallas_call(
        functools.partial(_bn_kernel, N),
        out_shape=jax.ShapeDtypeStruct((N, D), jnp.float32),
    )(S, deg, x, bias[None, :], gamma[None, :], beta[None, :])
    return out


# speedup vs baseline: 15.0951x; 1.3387x over previous
import functools

import jax
import jax.numpy as jnp
from jax import lax
from jax.experimental import pallas as pl
from jax.experimental.pallas import tpu as pltpu
from jax.experimental.pallas import tpu_sc as plsc

L = 16
CHUNK = 128
NC = 2
NS = 16
NW = NC * NS

_SC_PARAMS = pltpu.CompilerParams(needs_layout_passes=False)


def _make_deg_kernel(NP, n_chunks):
    DW = 128
    slice_rows = NP // NS
    cpt = n_chunks // NW
    mesh = plsc.VectorSubcoreMesh(core_axis_name="c", subcore_axis_name="s", num_cores=NC, num_subcores=NS)

    @functools.partial(
        pl.kernel,
        out_type=jax.ShapeDtypeStruct((NC, NP, DW), jnp.float32),
        mesh=mesh,
        compiler_params=_SC_PARAMS,
        scratch_types=[
            pltpu.VMEM((1, CHUNK), jnp.int32),
            pltpu.VMEM((1, CHUNK), jnp.float32),
            pltpu.VMEM((CHUNK, DW), jnp.float32),
            pltpu.VMEM_SHARED((NP, DW), jnp.float32),
        ],
    )
    def deg_kernel(colh, ewh, out, col_v, ew_v, wsrc, deg_sp):
        cid = lax.axis_index("c")
        sid = lax.axis_index("s")
        base = sid * slice_rows
        zero16 = jnp.zeros((16,), jnp.float32)

        def zrow(r, _):
            for j in range(DW // L):
                wsrc[r, pl.ds(L * j, L)] = zero16
            return 0
        lax.fori_loop(0, CHUNK, zrow, 0)
        for i in range(slice_rows // CHUNK):
            pltpu.sync_copy(wsrc, deg_sp.at[pl.ds(base + CHUNK * i, CHUNK)])
        plsc.subcore_barrier()

        def deg_chunk(c, _):
            ck = (cid * NS + sid) * cpt + c
            pltpu.sync_copy(colh.at[ck], col_v.at[0])
            pltpu.sync_copy(ewh.at[ck], ew_v.at[0])

            def splat_grp(s, _):
                e16 = ew_v[0, pl.ds(16 * s, 16)]
                for r2 in range(16):
                    wsrc[16 * s + r2, pl.ds(0, L)] = jnp.full(
                        (16,), e16[r2], jnp.float32)
                return 0
            lax.fori_loop(0, CHUNK // 16, splat_grp, 0)
            pltpu.sync_copy(wsrc, deg_sp.at[col_v.at[0]], add=True)
            return 0
        lax.fori_loop(0, cpt, deg_chunk, 0)
        plsc.subcore_barrier()
        pltpu.sync_copy(deg_sp.at[pl.ds(base, slice_rows)],
                        out.at[cid, pl.ds(base, slice_rows)])

    return deg_kernel


def _make_edge_kernel(NP, D, n_chunks):
    slice_rows = NP // NS
    cpt = n_chunks // NW
    mesh = plsc.VectorSubcoreMesh(core_axis_name="c", subcore_axis_name="s",
                                  num_cores=NC, num_subcores=NS)

    @functools.partial(
        pl.kernel,
        out_type=jax.ShapeDtypeStruct((NC, NP, D), jnp.float32),
        mesh=mesh,
        compiler_params=_SC_PARAMS,
        scratch_types=[
            pltpu.VMEM((3, CHUNK), jnp.int32),
            pltpu.VMEM((3, CHUNK), jnp.int32),
            pltpu.VMEM((CHUNK, D), jnp.float32),
            pltpu.VMEM((CHUNK, D), jnp.float32),
            pltpu.VMEM_SHARED((NP, D), jnp.float32),
            pltpu.SemaphoreType.DMA,
            pltpu.SemaphoreType.DMA,
            pltpu.SemaphoreType.DMA,
            pltpu.SemaphoreType.DMA,
        ],
    )
    def edge_kernel(g_hbm, edh, out,
                    ebufA, ebufB, rowsA, rowsB, S_sp,
                    gsemA, gsemB, ssemA, ssemB):
        cid = lax.axis_index("c")
        sid = lax.axis_index("s")
        base = sid * slice_rows
        tbase = (cid * NS + sid) * cpt
        zero16 = jnp.zeros((16,), jnp.float32)

        def issue_gather(ebuf, rows, sem):
            return pltpu.async_copy(g_hbm.at[ebuf.at[0]], rows, sem)

        def wait_gather(ebuf, rows, sem):
            pltpu.make_async_copy(g_hbm.at[ebuf.at[0]], rows, sem).wait()

        def issue_scatter(ebuf, rows, sem):
            return pltpu.async_copy(rows, S_sp.at[ebuf.at[1]], sem, add=True)

        def wait_scatter(ebuf, rows, sem):
            pltpu.make_async_copy(rows, S_sp.at[ebuf.at[1]], sem).wait()

        def scale(ebuf, rows):
            def scale_grp(s, _):
                e16 = plsc.bitcast(ebuf[2, pl.ds(16 * s, 16)], jnp.float32)
                for r2 in range(16):
                    r = 16 * s + r2
                    ws = e16[r2]
                    for j in range(D // L):
                        rows[r, pl.ds(L * j, L)] = (
                            rows[r, pl.ds(L * j, L)] * ws)
                return 0
            lax.fori_loop(0, CHUNK // 16, scale_grp, 0)

        def zrow(r, _):
            for j in range(D // L):
                rowsB[r, pl.ds(L * j, L)] = zero16
            return 0
        lax.fori_loop(0, CHUNK, zrow, 0)
        for i in range(slice_rows // CHUNK):
            pltpu.sync_copy(rowsB, S_sp.at[pl.ds(base + CHUNK * i, CHUNK)])
        plsc.subcore_barrier()

        pltpu.sync_copy(edh.at[tbase], ebufA)
        issue_gather(ebufA, rowsA, gsemA)
        issue_scatter(ebufA, rowsB, ssemB)

        def pair(t, _):
            c0 = tbase + 2 * t
            wait_scatter(ebufB, rowsB, ssemB)
            pltpu.sync_copy(edh.at[c0 + 1], ebufB)
            issue_gather(ebufB, rowsB, gsemB)
            wait_gather(ebufA, rowsA, gsemA)
            scale(ebufA, rowsA)
            issue_scatter(ebufA, rowsA, ssemA)
            wait_gather(ebufB, rowsB, gsemB)
            scale(ebufB, rowsB)
            issue_scatter(ebufB, rowsB, ssemB)

            @pl.when(2 * t + 2 < cpt)
            def _():
                wait_scatter(ebufA, rowsA, ssemA)
                pltpu.sync_copy(edh.at[c0 + 2], ebufA)
                issue_gather(ebufA, rowsA, gsemA)
            return 0
        lax.fori_loop(0, cpt // 2, pair, 0)

        if cpt % 2:
            wait_gather(ebufA, rowsA, gsemA)
            scale(ebufA, rowsA)
            pltpu.sync_copy(rowsA, S_sp.at[ebufA.at[1]], add=True)
        else:
            wait_scatter(ebufA, rowsA, ssemA)
        wait_scatter(ebufB, rowsB, ssemB)
        plsc.subcore_barrier()

        pltpu.sync_copy(S_sp.at[pl.ds(base, slice_rows)],
                        out.at[cid, pl.ds(base, slice_rows)])

    return edge_kernel


def _mm_kernel(deg_ref, x_ref, w_ref, o_ref):
    d = deg_ref[0, :, 0:1] + deg_ref[1, :, 0:1]
    dis = jnp.where(d > 0, lax.rsqrt(d), 0.0)
    o_ref[...] = dis * jnp.dot(x_ref[...], w_ref[...],
                               preferred_element_type=jnp.float32)


def _bn_kernel(N, s_ref, deg_ref, x_ref, b_ref, g_ref, be_ref, o_ref):
    d = deg_ref[0, :N, 0:1] + deg_ref[1, :N, 0:1]
    dis = jnp.where(d > 0, lax.rsqrt(d), 0.0)
    agg = dis * (s_ref[0, :N, :] + s_ref[1, :N, :]) + b_ref[...]
    mean = jnp.mean(agg, axis=0, keepdims=True)
    var = jnp.mean((agg - mean) ** 2, axis=0, keepdims=True)
    bn = (agg - mean) * lax.rsqrt(var + 1e-5) * g_ref[...] + be_ref[...]
    o_ref[...] = jnp.maximum(bn, 0.0) + x_ref[...]


def kernel(x, edge_index, edge_weight, W, bias, gamma, beta):
    N, D = x.shape
    E = edge_weight.shape[0]
    EA = E + N
    n_chunks = ((EA + CHUNK * NW - 1) // (CHUNK * NW)) * NW
    EP = n_chunks * CHUNK
    pad = EP - EA
    NP = ((N + NW * 8 - 1) // (NW * 8)) * (NW * 8)

    loop = jnp.arange(N, dtype=edge_index.dtype)
    zpad_i = jnp.zeros((pad,), edge_index.dtype)
    rowa = jnp.concatenate([edge_index[0], loop, zpad_i]).reshape(n_chunks,
                                                                  CHUNK)
    cola = jnp.concatenate([edge_index[1], loop, zpad_i]).reshape(n_chunks,
                                                                  CHUNK)
    ewa = jnp.concatenate([edge_weight, jnp.ones((N,), x.dtype),
                           jnp.zeros((pad,), x.dtype)]).reshape(n_chunks,
                                                                CHUNK)
    edata = jnp.stack(
        [rowa, cola, lax.bitcast_convert_type(ewa, jnp.int32)], axis=1)

    deg = _make_deg_kernel(NP, n_chunks)(cola, ewa)

    RB = 1000
    g = pl.pallas_call(
        _mm_kernel,
        grid=(N // RB,),
        in_specs=[pl.BlockSpec((NC, RB, 128), lambda i: (0, i, 0)),
                  pl.BlockSpec((RB, D), lambda i: (i, 0)),
                  pl.BlockSpec((D, D), lambda i: (0, 0))],
        out_specs=pl.BlockSpec((RB, D), lambda i: (i, 0)),
        out_shape=jax.ShapeDtypeStruct((N, D), jnp.float32),
    )(deg, x, W)

    S = _make_edge_kernel(NP, D, n_chunks)(g, edata)

    out = pl.pallas_call(
        functools.partial(_bn_kernel, N),
        out_shape=jax.ShapeDtypeStruct((N, D), jnp.float32),
    )(S, deg, x, bias[None, :], gamma[None, :], beta[None, :])
    return out


# speedup vs baseline: 19.5736x; 1.2967x over previous
import functools

import jax
import jax.numpy as jnp
from jax import lax
from jax.experimental import pallas as pl
from jax.experimental.pallas import tpu as pltpu
from jax.experimental.pallas import tpu_sc as plsc

L = 16
CHUNK = 128
NC = 2
NS = 16
NW = NC * NS

_SC_PARAMS = pltpu.CompilerParams(needs_layout_passes=False)


def _make_deg_kernel(NP, n_chunks):
    slice_rows = NP // NS
    cpt = n_chunks // NW
    mesh = plsc.VectorSubcoreMesh(core_axis_name="c", subcore_axis_name="s",
                                  num_cores=NC, num_subcores=NS)

    @functools.partial(
        pl.kernel,
        out_type=jax.ShapeDtypeStruct((NC, NP), jnp.float32),
        mesh=mesh,
        compiler_params=_SC_PARAMS,
        scratch_types=[
            pltpu.VMEM((3, CHUNK), jnp.int32),
            pltpu.VMEM((3, CHUNK), jnp.int32),
            pltpu.VMEM((NP,), jnp.float32),
            pltpu.VMEM((NS, slice_rows), jnp.float32),
            pltpu.VMEM((slice_rows,), jnp.float32),
            pltpu.VMEM_SHARED((NS, NP), jnp.float32),
            pltpu.SemaphoreType.DMA,
            pltpu.SemaphoreType.DMA,
        ],
    )
    def deg_kernel(edh, out, ebufA, ebufB, deg_local, red, acc, deg_stage,
                   isemA, isemB):
        cid = lax.axis_index("c")
        sid = lax.axis_index("s")
        base = sid * slice_rows
        tbase = (cid * NS + sid) * cpt

        def zloc(r, _):
            deg_local[pl.ds(16 * r, 16)] = jnp.zeros((16,), jnp.float32)
            return 0
        lax.fori_loop(0, NP // 16, zloc, 0)

        def process(ebuf):
            for s in range(CHUNK // 16):
                col16 = ebuf[1, pl.ds(16 * s, 16)]
                ew16 = plsc.bitcast(ebuf[2, pl.ds(16 * s, 16)], jnp.float32)
                plsc.addupdate_scatter(deg_local, [col16], ew16)

        pltpu.async_copy(edh.at[tbase], ebufA, isemA)

        def pair(t, _):
            c0 = tbase + 2 * t
            pltpu.async_copy(edh.at[c0 + 1], ebufB, isemB)
            pltpu.make_async_copy(edh.at[c0], ebufA, isemA).wait()
            process(ebufA)

            @pl.when(2 * t + 2 < cpt)
            def _():
                pltpu.async_copy(edh.at[c0 + 2], ebufA, isemA)
            pltpu.make_async_copy(edh.at[c0 + 1], ebufB, isemB).wait()
            process(ebufB)
            return 0
        lax.fori_loop(0, cpt // 2, pair, 0)
        if cpt % 2:
            pltpu.make_async_copy(edh.at[tbase], ebufA, isemA).wait()
            process(ebufA)

        pltpu.sync_copy(deg_local, deg_stage.at[sid])
        plsc.subcore_barrier()
        for k in range(NS):
            pltpu.sync_copy(deg_stage.at[k, pl.ds(base, slice_rows)],
                            red.at[k])

        def radd(j, _):
            v = red[0, pl.ds(16 * j, 16)]
            for k in range(1, NS):
                v = v + red[k, pl.ds(16 * j, 16)]
            acc[pl.ds(16 * j, 16)] = v
            return 0
        lax.fori_loop(0, slice_rows // 16, radd, 0)
        pltpu.sync_copy(acc, out.at[cid, pl.ds(base, slice_rows)])

    return deg_kernel


def _make_edge_kernel(NP, D, n_chunks):
    slice_rows = NP // NS
    cpt = n_chunks // NW
    mesh = plsc.VectorSubcoreMesh(core_axis_name="c", subcore_axis_name="s",
                                  num_cores=NC, num_subcores=NS)

    @functools.partial(
        pl.kernel,
        out_type=jax.ShapeDtypeStruct((NC, NP, D), jnp.float32),
        mesh=mesh,
        compiler_params=_SC_PARAMS,
        scratch_types=[
            pltpu.VMEM((3, CHUNK), jnp.int32),
            pltpu.VMEM((3, CHUNK), jnp.int32),
            pltpu.VMEM((CHUNK, D), jnp.float32),
            pltpu.VMEM((CHUNK, D), jnp.float32),
            pltpu.VMEM_SHARED((NP, D), jnp.float32),
            pltpu.SemaphoreType.DMA,
            pltpu.SemaphoreType.DMA,
            pltpu.SemaphoreType.DMA,
            pltpu.SemaphoreType.DMA,
        ],
    )
    def edge_kernel(g_hbm, edh, out,
                    ebufA, ebufB, rowsA, rowsB, S_sp,
                    gsemA, gsemB, ssemA, ssemB):
        cid = lax.axis_index("c")
        sid = lax.axis_index("s")
        base = sid * slice_rows
        tbase = (cid * NS + sid) * cpt
        zero16 = jnp.zeros((16,), jnp.float32)

        def issue_gather(ebuf, rows, sem):
            return pltpu.async_copy(g_hbm.at[ebuf.at[0]], rows, sem)

        def wait_gather(ebuf, rows, sem):
            pltpu.make_async_copy(g_hbm.at[ebuf.at[0]], rows, sem).wait()

        def issue_scatter(ebuf, rows, sem):
            return pltpu.async_copy(rows, S_sp.at[ebuf.at[1]], sem, add=True)

        def wait_scatter(ebuf, rows, sem):
            pltpu.make_async_copy(rows, S_sp.at[ebuf.at[1]], sem).wait()

        def scale(ebuf, rows):
            def scale_grp(s, _):
                e16 = plsc.bitcast(ebuf[2, pl.ds(16 * s, 16)], jnp.float32)
                for r2 in range(16):
                    r = 16 * s + r2
                    ws = e16[r2]
                    for j in range(D // L):
                        rows[r, pl.ds(L * j, L)] = (
                            rows[r, pl.ds(L * j, L)] * ws)
                return 0
            lax.fori_loop(0, CHUNK // 16, scale_grp, 0)

        def zrow(r, _):
            for j in range(D // L):
                rowsB[r, pl.ds(L * j, L)] = zero16
            return 0
        lax.fori_loop(0, CHUNK, zrow, 0)
        for i in range(slice_rows // CHUNK):
            pltpu.sync_copy(rowsB, S_sp.at[pl.ds(base + CHUNK * i, CHUNK)])
        plsc.subcore_barrier()

        pltpu.sync_copy(edh.at[tbase], ebufA)
        issue_gather(ebufA, rowsA, gsemA)
        issue_scatter(ebufA, rowsB, ssemB)

        def pair(t, _):
            c0 = tbase + 2 * t
            wait_scatter(ebufB, rowsB, ssemB)
            pltpu.sync_copy(edh.at[c0 + 1], ebufB)
            issue_gather(ebufB, rowsB, gsemB)
            wait_gather(ebufA, rowsA, gsemA)
            scale(ebufA, rowsA)
            issue_scatter(ebufA, rowsA, ssemA)
            wait_gather(ebufB, rowsB, gsemB)
            scale(ebufB, rowsB)
            issue_scatter(ebufB, rowsB, ssemB)

            @pl.when(2 * t + 2 < cpt)
            def _():
                wait_scatter(ebufA, rowsA, ssemA)
                pltpu.sync_copy(edh.at[c0 + 2], ebufA)
                issue_gather(ebufA, rowsA, gsemA)
            return 0
        lax.fori_loop(0, cpt // 2, pair, 0)

        if cpt % 2:
            wait_gather(ebufA, rowsA, gsemA)
            scale(ebufA, rowsA)
            pltpu.sync_copy(rowsA, S_sp.at[ebufA.at[1]], add=True)
        else:
            wait_scatter(ebufA, rowsA, ssemA)
        wait_scatter(ebufB, rowsB, ssemB)
        plsc.subcore_barrier()

        pltpu.sync_copy(S_sp.at[pl.ds(base, slice_rows)],
                        out.at[cid, pl.ds(base, slice_rows)])

    return edge_kernel


def _mm_kernel(deg_ref, x_ref, w_ref, o_ref):
    d = deg_ref[0] + deg_ref[1]
    dis = jnp.where(d > 0, lax.rsqrt(d), 0.0)
    o_ref[...] = dis * jnp.dot(x_ref[...], w_ref[...],
                               preferred_element_type=jnp.float32)


def _bn_kernel(N, s_ref, deg_ref, x_ref, b_ref, g_ref, be_ref, o_ref):
    d = deg_ref[0, :N] + deg_ref[1, :N]
    dis = jnp.where(d > 0, lax.rsqrt(d), 0.0)
    agg = dis * (s_ref[0, :N, :] + s_ref[1, :N, :]) + b_ref[...]
    mean = jnp.mean(agg, axis=0, keepdims=True)
    var = jnp.mean((agg - mean) ** 2, axis=0, keepdims=True)
    bn = (agg - mean) * lax.rsqrt(var + 1e-5) * g_ref[...] + be_ref[...]
    o_ref[...] = jnp.maximum(bn, 0.0) + x_ref[...]


def kernel(x, edge_index, edge_weight, W, bias, gamma, beta):
    N, D = x.shape
    E = edge_weight.shape[0]
    EA = E + N
    n_chunks = ((EA + CHUNK * NW - 1) // (CHUNK * NW)) * NW
    EP = n_chunks * CHUNK
    pad = EP - EA
    NP = ((N + NW * 8 - 1) // (NW * 8)) * (NW * 8)

    loop = jnp.arange(N, dtype=edge_index.dtype)
    zpad_i = jnp.zeros((pad,), edge_index.dtype)
    rowa = jnp.concatenate([edge_index[0], loop, zpad_i]).reshape(n_chunks,
                                                                  CHUNK)
    cola = jnp.concatenate([edge_index[1], loop, zpad_i]).reshape(n_chunks,
                                                                  CHUNK)
    ewa = jnp.concatenate([edge_weight, jnp.ones((N,), x.dtype),
                           jnp.zeros((pad,), x.dtype)]).reshape(n_chunks,
                                                                CHUNK)
    edata = jnp.stack(
        [rowa, cola, lax.bitcast_convert_type(ewa, jnp.int32)], axis=1)

    deg = _make_deg_kernel(NP, n_chunks)(edata)
    deg3 = deg[:, :, None]

    RB = 1000
    g = pl.pallas_call(
        _mm_kernel,
        grid=(N // RB,),
        in_specs=[pl.BlockSpec((NC, RB, 1), lambda i: (0, i, 0)),
                  pl.BlockSpec((RB, D), lambda i: (i, 0)),
                  pl.BlockSpec((D, D), lambda i: (0, 0))],
        out_specs=pl.BlockSpec((RB, D), lambda i: (i, 0)),
        out_shape=jax.ShapeDtypeStruct((N, D), jnp.float32),
    )(deg3, x, W)

    S = _make_edge_kernel(NP, D, n_chunks)(g, edata)

    out = pl.pallas_call(
        functools.partial(_bn_kernel, N),
        out_shape=jax.ShapeDtypeStruct((N, D), jnp.float32),
    )(S, deg3, x, bias[None, :], gamma[None, :], beta[None, :])
    return out


# speedup vs baseline: 20.1944x; 1.0317x over previous
import functools

import jax
import jax.numpy as jnp
from jax import lax
from jax.experimental import pallas as pl
from jax.experimental.pallas import tpu as pltpu
from jax.experimental.pallas import tpu_sc as plsc

L = 16
CHUNK = 128
NC = 2
NS = 16
NW = NC * NS

_SC_PARAMS = pltpu.CompilerParams(needs_layout_passes=False)


def _make_deg_kernel(NP, n_chunks):
    slice_rows = NP // NS
    cpt = n_chunks // NW
    mesh = plsc.VectorSubcoreMesh(core_axis_name="c", subcore_axis_name="s",
                                  num_cores=NC, num_subcores=NS)

    @functools.partial(
        pl.kernel,
        out_type=jax.ShapeDtypeStruct((NC, NP), jnp.float32),
        mesh=mesh,
        compiler_params=_SC_PARAMS,
        scratch_types=[
            pltpu.VMEM((3, CHUNK), jnp.int32),
            pltpu.VMEM((3, CHUNK), jnp.int32),
            pltpu.VMEM((NP,), jnp.float32),
            pltpu.VMEM((NS, slice_rows), jnp.float32),
            pltpu.VMEM((slice_rows,), jnp.float32),
            pltpu.VMEM_SHARED((NS, NP), jnp.float32),
            pltpu.SemaphoreType.DMA,
            pltpu.SemaphoreType.DMA,
        ],
    )
    def deg_kernel(edh, out, ebufA, ebufB, deg_local, red, acc, deg_stage,
                   isemA, isemB):
        cid = lax.axis_index("c")
        sid = lax.axis_index("s")
        base = sid * slice_rows
        tbase = (cid * NS + sid) * cpt

        def zloc(r, _):
            deg_local[pl.ds(16 * r, 16)] = jnp.zeros((16,), jnp.float32)
            return 0
        lax.fori_loop(0, NP // 16, zloc, 0)

        def process(ebuf):
            for s in range(CHUNK // 16):
                col16 = ebuf[1, pl.ds(16 * s, 16)]
                ew16 = plsc.bitcast(ebuf[2, pl.ds(16 * s, 16)], jnp.float32)
                plsc.addupdate_scatter(deg_local, [col16], ew16)

        pltpu.async_copy(edh.at[tbase], ebufA, isemA)

        def pair(t, _):
            c0 = tbase + 2 * t
            pltpu.async_copy(edh.at[c0 + 1], ebufB, isemB)
            pltpu.make_async_copy(edh.at[c0], ebufA, isemA).wait()
            process(ebufA)

            @pl.when(2 * t + 2 < cpt)
            def _():
                pltpu.async_copy(edh.at[c0 + 2], ebufA, isemA)
            pltpu.make_async_copy(edh.at[c0 + 1], ebufB, isemB).wait()
            process(ebufB)
            return 0
        lax.fori_loop(0, cpt // 2, pair, 0)
        if cpt % 2:
            pltpu.make_async_copy(edh.at[tbase], ebufA, isemA).wait()
            process(ebufA)

        pltpu.sync_copy(deg_local, deg_stage.at[sid])
        plsc.subcore_barrier()
        for k in range(NS):
            pltpu.sync_copy(deg_stage.at[k, pl.ds(base, slice_rows)],
                            red.at[k])

        def radd(j, _):
            v = red[0, pl.ds(16 * j, 16)]
            for k in range(1, NS):
                v = v + red[k, pl.ds(16 * j, 16)]
            acc[pl.ds(16 * j, 16)] = v
            return 0
        lax.fori_loop(0, slice_rows // 16, radd, 0)
        pltpu.sync_copy(acc, out.at[cid, pl.ds(base, slice_rows)])

    return deg_kernel


def _make_edge_kernel(NP, D, n_chunks):
    slice_rows = NP // NS
    cpt = n_chunks // NW
    mesh = plsc.VectorSubcoreMesh(core_axis_name="c", subcore_axis_name="s",
                                  num_cores=NC, num_subcores=NS)

    @functools.partial(
        pl.kernel,
        out_type=jax.ShapeDtypeStruct((NC, NP, D), jnp.float32),
        mesh=mesh,
        compiler_params=_SC_PARAMS,
        scratch_types=[
            pltpu.VMEM((3, CHUNK), jnp.int32),
            pltpu.VMEM((3, CHUNK), jnp.int32),
            pltpu.VMEM((CHUNK, D), jnp.float32),
            pltpu.VMEM((CHUNK, D), jnp.float32),
            pltpu.VMEM_SHARED((NP, D), jnp.float32),
            pltpu.SemaphoreType.DMA,
            pltpu.SemaphoreType.DMA,
            pltpu.SemaphoreType.DMA,
            pltpu.SemaphoreType.DMA,
        ],
    )
    def edge_kernel(g_hbm, edh, out,
                    ebufA, ebufB, rowsA, rowsB, S_sp,
                    gsemA, gsemB, ssemA, ssemB):
        cid = lax.axis_index("c")
        sid = lax.axis_index("s")
        base = sid * slice_rows
        wid = cid * NS + sid
        zero16 = jnp.zeros((16,), jnp.float32)

        def issue_gather(ebuf, rows, sem):
            return pltpu.async_copy(g_hbm.at[ebuf.at[0]], rows, sem)

        def wait_gather(ebuf, rows, sem):
            pltpu.make_async_copy(g_hbm.at[ebuf.at[0]], rows, sem).wait()

        def issue_scatter(ebuf, rows, sem):
            return pltpu.async_copy(rows, S_sp.at[ebuf.at[1]], sem, add=True)

        def wait_scatter(ebuf, rows, sem):
            pltpu.make_async_copy(rows, S_sp.at[ebuf.at[1]], sem).wait()

        def scale(ebuf, rows):
            def scale_grp(s, _):
                e16 = plsc.bitcast(ebuf[2, pl.ds(16 * s, 16)], jnp.float32)
                for r2 in range(16):
                    r = 16 * s + r2
                    ws = e16[r2]
                    for j in range(D // L):
                        rows[r, pl.ds(L * j, L)] = (
                            rows[r, pl.ds(L * j, L)] * ws)
                return 0
            lax.fori_loop(0, CHUNK // 16, scale_grp, 0)

        def zrow(r, _):
            for j in range(D // L):
                rowsB[r, pl.ds(L * j, L)] = zero16
            return 0
        lax.fori_loop(0, CHUNK, zrow, 0)
        for i in range(slice_rows // CHUNK):
            pltpu.sync_copy(rowsB, S_sp.at[pl.ds(base + CHUNK * i, CHUNK)])
        plsc.subcore_barrier()

        pltpu.sync_copy(edh.at[wid], ebufA)
        issue_gather(ebufA, rowsA, gsemA)
        issue_scatter(ebufA, rowsB, ssemB)

        def pair(t, _):
            cg0 = wid + (2 * t) * NW
            wait_scatter(ebufB, rowsB, ssemB)
            pltpu.sync_copy(edh.at[cg0 + NW], ebufB)
            issue_gather(ebufB, rowsB, gsemB)
            wait_gather(ebufA, rowsA, gsemA)
            scale(ebufA, rowsA)
            issue_scatter(ebufA, rowsA, ssemA)
            wait_gather(ebufB, rowsB, gsemB)
            scale(ebufB, rowsB)
            issue_scatter(ebufB, rowsB, ssemB)

            @pl.when(2 * t + 2 < cpt)
            def _():
                wait_scatter(ebufA, rowsA, ssemA)
                pltpu.sync_copy(edh.at[cg0 + 2 * NW], ebufA)
                issue_gather(ebufA, rowsA, gsemA)
            return 0
        lax.fori_loop(0, cpt // 2, pair, 0)

        if cpt % 2:
            wait_gather(ebufA, rowsA, gsemA)
            scale(ebufA, rowsA)
            pltpu.sync_copy(rowsA, S_sp.at[ebufA.at[1]], add=True)
        else:
            wait_scatter(ebufA, rowsA, ssemA)
        wait_scatter(ebufB, rowsB, ssemB)
        plsc.subcore_barrier()

        pltpu.sync_copy(S_sp.at[pl.ds(base, slice_rows)],
                        out.at[cid, pl.ds(base, slice_rows)])

    return edge_kernel


def _mm_kernel(deg_ref, x_ref, w_ref, o_ref):
    d = deg_ref[0] + deg_ref[1]
    dis = jnp.where(d > 0, lax.rsqrt(d), 0.0)
    o_ref[...] = dis * jnp.dot(x_ref[...], w_ref[...],
                               preferred_element_type=jnp.float32)


def _bn_kernel(N, s_ref, deg_ref, x_ref, b_ref, g_ref, be_ref, o_ref):
    d = deg_ref[0, :N] + deg_ref[1, :N]
    dis = jnp.where(d > 0, lax.rsqrt(d), 0.0)
    agg = dis * (s_ref[0, :N, :] + s_ref[1, :N, :]) + b_ref[...]
    mean = jnp.mean(agg, axis=0, keepdims=True)
    var = jnp.mean((agg - mean) ** 2, axis=0, keepdims=True)
    bn = (agg - mean) * lax.rsqrt(var + 1e-5) * g_ref[...] + be_ref[...]
    o_ref[...] = jnp.maximum(bn, 0.0) + x_ref[...]


def kernel(x, edge_index, edge_weight, W, bias, gamma, beta):
    N, D = x.shape
    E = edge_weight.shape[0]
    EA = E + N
    n_chunks = ((EA + CHUNK * NW - 1) // (CHUNK * NW)) * NW
    EP = n_chunks * CHUNK
    pad = EP - EA
    NP = ((N + NW * 8 - 1) // (NW * 8)) * (NW * 8)

    loop = jnp.arange(N, dtype=edge_index.dtype)
    zpad_i = jnp.zeros((pad,), edge_index.dtype)
    rowa = jnp.concatenate([edge_index[0], loop, zpad_i]).reshape(n_chunks,
                                                                  CHUNK)
    cola = jnp.concatenate([edge_index[1], loop, zpad_i]).reshape(n_chunks,
                                                                  CHUNK)
    ewa = jnp.concatenate([edge_weight, jnp.ones((N,), x.dtype),
                           jnp.zeros((pad,), x.dtype)]).reshape(n_chunks,
                                                                CHUNK)
    edata = jnp.stack(
        [rowa, cola, lax.bitcast_convert_type(ewa, jnp.int32)], axis=1)

    deg = _make_deg_kernel(NP, n_chunks)(edata)
    deg3 = deg[:, :, None]

    RB = 1000
    g = pl.pallas_call(
        _mm_kernel,
        grid=(N // RB,),
        in_specs=[pl.BlockSpec((NC, RB, 1), lambda i: (0, i, 0)),
                  pl.BlockSpec((RB, D), lambda i: (i, 0)),
                  pl.BlockSpec((D, D), lambda i: (0, 0))],
        out_specs=pl.BlockSpec((RB, D), lambda i: (i, 0)),
        out_shape=jax.ShapeDtypeStruct((N, D), jnp.float32),
    )(deg3, x, W)

    S = _make_edge_kernel(NP, D, n_chunks)(g, edata)

    out = pl.pallas_call(
        functools.partial(_bn_kernel, N),
        out_shape=jax.ShapeDtypeStruct((N, D), jnp.float32),
    )(S, deg3, x, bias[None, :], gamma[None, :], beta[None, :])
    return out


# speedup vs baseline: 28.3890x; 1.4058x over previous
import functools

import jax
import jax.numpy as jnp
from jax import lax
from jax.experimental import pallas as pl
from jax.experimental.pallas import tpu as pltpu
from jax.experimental.pallas import tpu_sc as plsc

L = 16
CHUNK = 128
NC = 2
NS = 16
NW = NC * NS

_SC_PARAMS = pltpu.CompilerParams(needs_layout_passes=False)


def _make_deg_kernel(NP, n_chunks):
    slice_rows = NP // NS
    cpt = n_chunks // NW
    rem = n_chunks - cpt * NW
    assert cpt % 2 == 0
    mesh = plsc.VectorSubcoreMesh(core_axis_name="c", subcore_axis_name="s",
                                  num_cores=NC, num_subcores=NS)

    @functools.partial(
        pl.kernel,
        out_type=jax.ShapeDtypeStruct((NC, NP), jnp.float32),
        mesh=mesh,
        compiler_params=_SC_PARAMS,
        scratch_types=[
            pltpu.VMEM((1, CHUNK), jnp.int32),
            pltpu.VMEM((1, CHUNK), jnp.int32),
            pltpu.VMEM((1, CHUNK), jnp.float32),
            pltpu.VMEM((1, CHUNK), jnp.float32),
            pltpu.VMEM((NP,), jnp.float32),
            pltpu.VMEM((NS, slice_rows), jnp.float32),
            pltpu.VMEM((slice_rows,), jnp.float32),
            pltpu.VMEM_SHARED((NS, NP), jnp.float32),
            pltpu.SemaphoreType.DMA,
            pltpu.SemaphoreType.DMA,
        ],
    )
    def deg_kernel(ei3, ew2, out, cbufA, cbufB, wbufA, wbufB,
                   deg_local, red, acc, deg_stage, isemA, isemB):
        cid = lax.axis_index("c")
        sid = lax.axis_index("s")
        base = sid * slice_rows
        wid = cid * NS + sid

        def zloc(r, _):
            deg_local[pl.ds(16 * r, 16)] = jnp.zeros((16,), jnp.float32)
            return 0
        lax.fori_loop(0, NP // 16, zloc, 0)

        def issue(ck, cbuf, wbuf, sem):
            pltpu.async_copy(ei3.at[1, ck], cbuf.at[0], sem)
            pltpu.async_copy(ew2.at[ck], wbuf.at[0], sem)

        def wait(cbuf, wbuf, sem):
            pltpu.make_async_copy(ei3.at[1, 0], cbuf.at[0], sem).wait()
            pltpu.make_async_copy(ew2.at[0], wbuf.at[0], sem).wait()

        def process(cbuf, wbuf):
            for s in range(CHUNK // 16):
                col16 = cbuf[0, pl.ds(16 * s, 16)]
                ew16 = wbuf[0, pl.ds(16 * s, 16)]
                plsc.addupdate_scatter(deg_local, [col16], ew16)

        issue(wid, cbufA, wbufA, isemA)

        def pair(t, _):
            cg0 = wid + (2 * t) * NW
            issue(cg0 + NW, cbufB, wbufB, isemB)
            wait(cbufA, wbufA, isemA)
            process(cbufA, wbufA)

            @pl.when(2 * t + 2 < cpt)
            def _():
                issue(cg0 + 2 * NW, cbufA, wbufA, isemA)
            wait(cbufB, wbufB, isemB)
            process(cbufB, wbufB)
            return 0
        lax.fori_loop(0, cpt // 2, pair, 0)

        if rem:
            @pl.when(wid < rem)
            def _():
                issue(cpt * NW + wid, cbufA, wbufA, isemA)
                wait(cbufA, wbufA, isemA)
                process(cbufA, wbufA)

        pltpu.sync_copy(deg_local, deg_stage.at[sid])
        plsc.subcore_barrier()
        for k in range(NS):
            pltpu.sync_copy(deg_stage.at[k, pl.ds(base, slice_rows)],
                            red.at[k])

        def radd(j, _):
            v = red[0, pl.ds(16 * j, 16)]
            for k in range(1, NS):
                v = v + red[k, pl.ds(16 * j, 16)]
            acc[pl.ds(16 * j, 16)] = v + 0.5
            return 0
        lax.fori_loop(0, slice_rows // 16, radd, 0)
        pltpu.sync_copy(acc, out.at[cid, pl.ds(base, slice_rows)])

    return deg_kernel


def _make_edge_kernel(NP, D, n_chunks):
    slice_rows = NP // NS
    cpt = n_chunks // NW
    rem = n_chunks - cpt * NW
    assert cpt % 2 == 0
    mesh = plsc.VectorSubcoreMesh(core_axis_name="c", subcore_axis_name="s",
                                  num_cores=NC, num_subcores=NS)

    @functools.partial(
        pl.kernel,
        out_type=jax.ShapeDtypeStruct((NC, NP, D), jnp.float32),
        mesh=mesh,
        compiler_params=_SC_PARAMS,
        scratch_types=[
            pltpu.VMEM((2, CHUNK), jnp.int32),
            pltpu.VMEM((2, CHUNK), jnp.int32),
            pltpu.VMEM((1, CHUNK), jnp.float32),
            pltpu.VMEM((1, CHUNK), jnp.float32),
            pltpu.VMEM((CHUNK, D), jnp.float32),
            pltpu.VMEM((CHUNK, D), jnp.float32),
            pltpu.VMEM_SHARED((NP, D), jnp.float32),
            pltpu.SemaphoreType.DMA,
            pltpu.SemaphoreType.DMA,
            pltpu.SemaphoreType.DMA,
            pltpu.SemaphoreType.DMA,
            pltpu.SemaphoreType.DMA,
            pltpu.SemaphoreType.DMA,
        ],
    )
    def edge_kernel(g_hbm, ei3, ew2, out,
                    ibufA, ibufB, wbufA, wbufB, rowsA, rowsB, S_sp,
                    isemA, isemB, gsemA, gsemB, ssemA, ssemB):
        cid = lax.axis_index("c")
        sid = lax.axis_index("s")
        base = sid * slice_rows
        wid = cid * NS + sid
        zero16 = jnp.zeros((16,), jnp.float32)

        def issue_idx(ck, ibuf, wbuf, sem):
            pltpu.async_copy(ei3.at[0, ck], ibuf.at[0], sem)
            pltpu.async_copy(ei3.at[1, ck], ibuf.at[1], sem)
            pltpu.async_copy(ew2.at[ck], wbuf.at[0], sem)

        def wait_idx(ibuf, wbuf, sem):
            pltpu.make_async_copy(ei3.at[0, 0], ibuf.at[0], sem).wait()
            pltpu.make_async_copy(ei3.at[1, 0], ibuf.at[1], sem).wait()
            pltpu.make_async_copy(ew2.at[0], wbuf.at[0], sem).wait()

        def issue_gather(ibuf, rows, sem):
            pltpu.async_copy(g_hbm.at[ibuf.at[0]], rows, sem)

        def wait_gather(ibuf, rows, sem):
            pltpu.make_async_copy(g_hbm.at[ibuf.at[0]], rows, sem).wait()

        def issue_scatter(ibuf, rows, sem):
            pltpu.async_copy(rows, S_sp.at[ibuf.at[1]], sem, add=True)

        def wait_scatter(ibuf, rows, sem):
            pltpu.make_async_copy(rows, S_sp.at[ibuf.at[1]], sem).wait()

        def scale(wbuf, rows):
            def scale_grp(s, _):
                e16 = wbuf[0, pl.ds(16 * s, 16)]
                for r2 in range(16):
                    r = 16 * s + r2
                    ws = e16[r2]
                    for j in range(D // L):
                        rows[r, pl.ds(L * j, L)] = (
                            rows[r, pl.ds(L * j, L)] * ws)
                return 0
            lax.fori_loop(0, CHUNK // 16, scale_grp, 0)

        def zrow(r, _):
            for j in range(D // L):
                rowsB[r, pl.ds(L * j, L)] = zero16
            return 0
        lax.fori_loop(0, CHUNK, zrow, 0)
        for i in range(slice_rows // CHUNK):
            pltpu.sync_copy(rowsB, S_sp.at[pl.ds(base + CHUNK * i, CHUNK)])
        plsc.subcore_barrier()

        issue_idx(wid, ibufA, wbufA, isemA)
        wait_idx(ibufA, wbufA, isemA)
        issue_gather(ibufA, rowsA, gsemA)
        issue_scatter(ibufA, rowsB, ssemB)

        def pair(t, _):
            cg0 = wid + (2 * t) * NW
            wait_scatter(ibufB, rowsB, ssemB)
            issue_idx(cg0 + NW, ibufB, wbufB, isemB)
            wait_idx(ibufB, wbufB, isemB)
            issue_gather(ibufB, rowsB, gsemB)
            wait_gather(ibufA, rowsA, gsemA)
            scale(wbufA, rowsA)
            issue_scatter(ibufA, rowsA, ssemA)
            wait_gather(ibufB, rowsB, gsemB)
            scale(wbufB, rowsB)
            issue_scatter(ibufB, rowsB, ssemB)

            @pl.when(2 * t + 2 < cpt)
            def _():
                wait_scatter(ibufA, rowsA, ssemA)
                issue_idx(cg0 + 2 * NW, ibufA, wbufA, isemA)
                wait_idx(ibufA, wbufA, isemA)
                issue_gather(ibufA, rowsA, gsemA)
            return 0
        lax.fori_loop(0, cpt // 2, pair, 0)
        wait_scatter(ibufA, rowsA, ssemA)
        wait_scatter(ibufB, rowsB, ssemB)

        if rem:
            @pl.when(wid < rem)
            def _():
                ck = cpt * NW + wid
                issue_idx(ck, ibufA, wbufA, isemA)
                wait_idx(ibufA, wbufA, isemA)
                issue_gather(ibufA, rowsA, gsemA)
                wait_gather(ibufA, rowsA, gsemA)
                scale(wbufA, rowsA)
                pltpu.sync_copy(rowsA, S_sp.at[ibufA.at[1]], add=True)
        plsc.subcore_barrier()

        pltpu.sync_copy(S_sp.at[pl.ds(base, slice_rows)],
                        out.at[cid, pl.ds(base, slice_rows)])

    return edge_kernel


def _mm_kernel(deg_ref, x_ref, w_ref, o_ref):
    d = deg_ref[0] + deg_ref[1]
    dis = lax.rsqrt(d)
    o_ref[...] = dis * jnp.dot(x_ref[...], w_ref[...],
                               preferred_element_type=jnp.float32)


def _bn_kernel(N, s_ref, deg_ref, g_ref, x_ref, b_ref, ga_ref, be_ref,
               o_ref):
    d = deg_ref[0, :N] + deg_ref[1, :N]
    dis = lax.rsqrt(d)
    agg = dis * (s_ref[0, :N, :] + s_ref[1, :N, :] + g_ref[...]) + b_ref[...]
    mean = jnp.mean(agg, axis=0, keepdims=True)
    var = jnp.mean((agg - mean) ** 2, axis=0, keepdims=True)
    bn = (agg - mean) * lax.rsqrt(var + 1e-5) * ga_ref[...] + be_ref[...]
    o_ref[...] = jnp.maximum(bn, 0.0) + x_ref[...]


def kernel(x, edge_index, edge_weight, W, bias, gamma, beta):
    N, D = x.shape
    E = edge_weight.shape[0]
    assert E % CHUNK == 0
    n_chunks = E // CHUNK
    NP = ((N + NW * 8 - 1) // (NW * 8)) * (NW * 8)

    ei3 = edge_index.reshape(2, n_chunks, CHUNK)
    ew2 = edge_weight.reshape(n_chunks, CHUNK)

    deg = _make_deg_kernel(NP, n_chunks)(ei3, ew2)
    deg3 = deg[:, :, None]

    RB = 1000
    g = pl.pallas_call(
        _mm_kernel,
        grid=(N // RB,),
        in_specs=[pl.BlockSpec((NC, RB, 1), lambda i: (0, i, 0)),
                  pl.BlockSpec((RB, D), lambda i: (i, 0)),
                  pl.BlockSpec((D, D), lambda i: (0, 0))],
        out_specs=pl.BlockSpec((RB, D), lambda i: (i, 0)),
        out_shape=jax.ShapeDtypeStruct((N, D), jnp.float32),
    )(deg3, x, W)

    S = _make_edge_kernel(NP, D, n_chunks)(g, ei3, ew2)

    out = pl.pallas_call(
        functools.partial(_bn_kernel, N),
        out_shape=jax.ShapeDtypeStruct((N, D), jnp.float32),
    )(S, deg3, g, x, bias[None, :], gamma[None, :], beta[None, :])
    return out
